# baseline (speedup 1.0000x reference)
import functools

import jax
import jax.numpy as jnp
from jax import lax
from jax.experimental import pallas as pl
from jax.experimental.pallas import tpu as pltpu
from jax.experimental.pallas import tpu_sc as plsc

_N = 10000
_E = 320000
_C = 32
_R_MAX = 5.0
_INV_AVG = 1.0 / 32.0

_NC = 2
_NS = 16
_NW = _NC * _NS
_EPW = _E // _NW
_K = 1000
_G = _EPW // _K
_KS = 400
_GS = _EPW // _KS
_NPAD = 10240
_RPS = _NPAD // _NS

_sc_mesh = plsc.VectorSubcoreMesh(core_axis_name="c", subcore_axis_name="s")


@functools.partial(
    pl.kernel,
    out_type=(
        jax.ShapeDtypeStruct((_E, 8), jnp.float32),
        jax.ShapeDtypeStruct((_E, 8), jnp.float32),
    ),
    mesh=_sc_mesh,
    scratch_types=[
        pltpu.VMEM((_K,), jnp.int32),
        pltpu.VMEM((_K,), jnp.int32),
        pltpu.VMEM((_K, 8), jnp.float32),
        pltpu.VMEM((_K, 8), jnp.float32),
        pltpu.SemaphoreType.DMA,
        pltpu.SemaphoreType.DMA,
    ],
    compiler_params=pltpu.CompilerParams(use_tc_tiling_on_sc=False),
)
def _gather_positions(pos_hbm, snd_hbm, rcv_hbm, ps_hbm, pr_hbm,
                      idx_s, idx_r, bufs, bufr, sem_s, sem_r):
    wid = lax.axis_index("s") * _NC + lax.axis_index("c")
    base = wid * _EPW

    def body(g, carry):
        off = base + g * _K
        pltpu.sync_copy(snd_hbm.at[pl.ds(off, _K)], idx_s)
        pltpu.sync_copy(rcv_hbm.at[pl.ds(off, _K)], idx_r)
        cs = pltpu.async_copy(pos_hbm.at[idx_s], bufs, sem_s)
        cr = pltpu.async_copy(pos_hbm.at[idx_r], bufr, sem_r)
        cs.wait()
        cr.wait()
        pltpu.sync_copy(bufs, ps_hbm.at[pl.ds(off, _K)])
        pltpu.sync_copy(bufr, pr_hbm.at[pl.ds(off, _K)])
        return carry

    lax.fori_loop(0, _G, body, 0)


@functools.partial(
    pl.kernel,
    out_type=jax.ShapeDtypeStruct((_NC, _NPAD, _C), jnp.float32),
    mesh=_sc_mesh,
    scratch_types=[
        pltpu.VMEM((_KS,), jnp.int32),
        pltpu.VMEM((_KS,), jnp.int32),
        pltpu.VMEM((_KS, _C), jnp.float32),
        pltpu.VMEM((_KS // 8, 8, 128), jnp.float32),
        pltpu.VMEM((_RPS, _C), jnp.float32),
        pltpu.VMEM_SHARED((_NPAD, _C), jnp.float32),
        pltpu.SemaphoreType.DMA,
        pltpu.SemaphoreType.DMA,
    ],
    compiler_params=pltpu.CompilerParams(use_tc_tiling_on_sc=False),
)
def _scatter_messages(tbl_hbm, m_hbm, snd_hbm, rcv_hbm, out_hbm,
                      idx_s, idx_r, rows, mbuf, zbuf, acc, sem_g, sem_m):
    cid = lax.axis_index("c")
    sid = lax.axis_index("s")
    wid = sid * _NC + cid

    zero = jnp.zeros((16,), jnp.float32)

    def zb(i, carry):
        zbuf[i, pl.ds(0, 16)] = zero
        zbuf[i, pl.ds(16, 16)] = zero
        return carry

    lax.fori_loop(0, _RPS, zb, 0)
    pltpu.sync_copy(zbuf, acc.at[pl.ds(sid * _RPS, _RPS)])
    plsc.subcore_barrier()

    base = wid * _EPW

    def body(g, carry):
        off = base + g * _KS
        pltpu.sync_copy(snd_hbm.at[pl.ds(off, _KS)], idx_s)
        pltpu.sync_copy(rcv_hbm.at[pl.ds(off, _KS)], idx_r)
        cg = pltpu.async_copy(tbl_hbm.at[idx_s], rows, sem_g)
        cm = pltpu.async_copy(m_hbm.at[pl.ds(off // 8, _KS // 8)], mbuf, sem_m)
        cg.wait()
        cm.wait()

        def mul(t, c2):
            for s in range(8):
                rr = t * 8 + s
                rows[rr, pl.ds(0, 16)] = rows[rr, pl.ds(0, 16)] * mbuf[t, s, pl.ds(0, 16)]
                rows[rr, pl.ds(16, 16)] = rows[rr, pl.ds(16, 16)] * mbuf[t, s, pl.ds(16, 16)]
            return c2

        lax.fori_loop(0, _KS // 8, mul, 0)
        pltpu.sync_copy(rows, acc.at[idx_r], add=True)
        return carry

    lax.fori_loop(0, _GS, body, 0)
    plsc.subcore_barrier()
    pltpu.sync_copy(acc.at[pl.ds(sid * _RPS, _RPS)],
                    out_hbm.at[cid, pl.ds(sid * _RPS, _RPS)])


def _silu(x):
    return x / (1.0 + jnp.exp(-x))


def _prep_body(na_ref, we_ref, out_ref):
    out_ref[...] = jnp.dot(na_ref[...], we_ref[...],
                           preferred_element_type=jnp.float32)


_T = 6400


def _mlp_body(pst_ref, prt_ref, w1at, w2at, w3a, w1bt, w2bt, w3b,
              ma_ref, mb_ref):
    dt = prt_ref[...] - pst_ref[...]
    q = dt * dt
    ones18 = jnp.ones((1, 8), jnp.float32)
    r2t = jnp.dot(ones18, q, preferred_element_type=jnp.float32) + 1e-12
    rt = jnp.sqrt(r2t)
    x = rt * (1.0 / _R_MAX)
    x2 = x * x
    x4 = x2 * x2
    x5 = x4 * x
    env = 1.0 - 21.0 * x5 + 35.0 * x5 * x - 15.0 * x5 * x2
    env = jnp.where(x < 1.0, env, 0.0)
    scale = env * (jnp.sqrt(2.0 / _R_MAX) / rt)
    n = 1.0 + lax.broadcasted_iota(jnp.int32, (8, 1), 0).astype(jnp.float32)
    arg = n * ((jnp.pi / _R_MAX) * rt)
    ef = jnp.sin(arg) * scale

    ha = _silu(jnp.dot(w1at[...], ef, preferred_element_type=jnp.float32))
    ha = _silu(jnp.dot(w2at[...], ha, preferred_element_type=jnp.float32))
    ma = lax.dot_general(
        ha, w3a[...], (((0,), (0,)), ((), ())),
        preferred_element_type=jnp.float32)
    ma_ref[:, :, pl.ds(0, _C)] = ma.reshape(_T // 8, 8, _C)

    hb = _silu(jnp.dot(w1bt[...], ef, preferred_element_type=jnp.float32))
    hb = _silu(jnp.dot(w2bt[...], hb, preferred_element_type=jnp.float32))
    mb = lax.dot_general(
        hb, w3b[...], (((0,), (0,)), ((), ())),
        preferred_element_type=jnp.float32)
    mb_ref[:, :, pl.ds(0, _C)] = mb.reshape(_T // 8, 8, _C)


def _update_body(agg_ref, na_ref, wel_ref, ns_ref, out_ref):
    a = agg_ref[...]
    agg = a[0, :_N] + a[1, :_N]
    wel = jnp.dot(na_ref[...], wel_ref[...], preferred_element_type=jnp.float32)
    out_ref[...] = agg * _INV_AVG * wel + ns_ref[...]


def _final_body(aggb_ref, s1_ref, na_ref, welb_ref, wra_ref, wrb_ref, ae_ref,
                out_ref):
    a = aggb_ref[...]
    aggb = a[0, :_N] + a[1, :_N]
    na = na_ref[...]
    s1 = s1_ref[...]
    welb = jnp.dot(na, welb_ref[...], preferred_element_type=jnp.float32)
    s2 = aggb * _INV_AVG * welb + s1
    e0 = jnp.dot(na, ae_ref[...], preferred_element_type=jnp.float32)
    e1 = jnp.dot(s1, wra_ref[...], preferred_element_type=jnp.float32)
    e2 = jnp.dot(_silu(s2), wrb_ref[...], preferred_element_type=jnp.float32)
    out_ref[...] = jnp.sum(e0 + e1 + e2).reshape(1, 1)


def kernel(positions, node_attrs, W_embed, atomic_energies,
           Wr1_a, Wr2_a, Wr3_a, Welem_a, Wread_a,
           Wr1_b, Wr2_b, Wr3_b, Welem_b, Wread_b,
           edge_index, batch):
    sender = edge_index[0]
    receiver = edge_index[1]
    pos_pad = jnp.pad(positions, ((0, 0), (0, 5)))
    w3a = Wr3_a[:, :_C]
    w3b = Wr3_b[:, :_C]
    ae = atomic_energies.reshape(-1, 1)

    pos_s, pos_r = _gather_positions(pos_pad, sender, receiver)

    node_scalars = pl.pallas_call(
        _prep_body,
        out_shape=jax.ShapeDtypeStruct((_N, _C), jnp.float32),
    )(node_attrs, W_embed)

    pst = pos_s.T[:8]
    prt = pos_r.T[:8]
    w1at, w2at = Wr1_a.T, Wr2_a.T
    w1bt, w2bt = Wr1_b.T, Wr2_b.T
    nt = _E // _T
    full = lambda shape: pl.BlockSpec(shape, lambda i: (0, 0))
    m_a, m_b = pl.pallas_call(
        _mlp_body,
        grid=(nt,),
        in_specs=[
            pl.BlockSpec((8, _T), lambda i: (0, i)),
            pl.BlockSpec((8, _T), lambda i: (0, i)),
            full(w1at.shape), full(w2at.shape), full(w3a.shape),
            full(w1bt.shape), full(w2bt.shape), full(w3b.shape),
        ],
        out_specs=[
            pl.BlockSpec((_T // 8, 8, 128), lambda i: (i, 0, 0)),
            pl.BlockSpec((_T // 8, 8, 128), lambda i: (i, 0, 0)),
        ],
        out_shape=[
            jax.ShapeDtypeStruct((_E // 8, 8, 128), jnp.float32),
            jax.ShapeDtypeStruct((_E // 8, 8, 128), jnp.float32),
        ],
        compiler_params=pltpu.CompilerParams(
            fuse_transposed_lhs_in_matmul=True),
    )(pst, prt, w1at, w2at, w3a, w1bt, w2bt, w3b)

    agg_a = _scatter_messages(node_scalars, m_a, sender, receiver)

    s1 = pl.pallas_call(
        _update_body,
        out_shape=jax.ShapeDtypeStruct((_N, _C), jnp.float32),
    )(agg_a, node_attrs, Welem_a, node_scalars)

    agg_b = _scatter_messages(s1, m_b, sender, receiver)

    e = pl.pallas_call(
        _final_body,
        out_shape=jax.ShapeDtypeStruct((1, 1), jnp.float32),
    )(agg_b, s1, node_attrs, Welem_b, Wread_a, Wread_b, ae)

    return e[0]


# speedup vs baseline: 1.2988x; 1.2988x over previous
import functools

import jax
import jax.numpy as jnp
from jax import lax
from jax.experimental import pallas as pl
from jax.experimental.pallas import tpu as pltpu
from jax.experimental.pallas import tpu_sc as plsc

_N = 10000
_E = 320000
_C = 32
_R_MAX = 5.0
_INV_AVG = 1.0 / 32.0

_NC = 2
_NS = 16
_NW = _NC * _NS
_EPW = _E // _NW
_K = 1000
_G = _EPW // _K
_KS = 200
_GS = _EPW // _KS
_NPAD = 10240
_RPS = _NPAD // _NS

_sc_mesh = plsc.VectorSubcoreMesh(core_axis_name="c", subcore_axis_name="s")


@functools.partial(
    pl.kernel,
    out_type=(
        jax.ShapeDtypeStruct((_E, 8), jnp.float32),
        jax.ShapeDtypeStruct((_E, 8), jnp.float32),
    ),
    mesh=_sc_mesh,
    scratch_types=[
        pltpu.VMEM((_K,), jnp.int32),
        pltpu.VMEM((_K,), jnp.int32),
        pltpu.VMEM((_K, 8), jnp.float32),
        pltpu.VMEM((_K, 8), jnp.float32),
        pltpu.SemaphoreType.DMA,
        pltpu.SemaphoreType.DMA,
    ],
    compiler_params=pltpu.CompilerParams(use_tc_tiling_on_sc=False),
)
def _gather_positions(pos_hbm, snd_hbm, rcv_hbm, ps_hbm, pr_hbm,
                      idx_s, idx_r, bufs, bufr, sem_s, sem_r):
    wid = lax.axis_index("s") * _NC + lax.axis_index("c")
    base = wid * _EPW

    def body(g, carry):
        off = base + g * _K
        pltpu.sync_copy(snd_hbm.at[pl.ds(off, _K)], idx_s)
        pltpu.sync_copy(rcv_hbm.at[pl.ds(off, _K)], idx_r)
        cs = pltpu.async_copy(pos_hbm.at[idx_s], bufs, sem_s)
        cr = pltpu.async_copy(pos_hbm.at[idx_r], bufr, sem_r)
        cs.wait()
        cr.wait()
        pltpu.sync_copy(bufs, ps_hbm.at[pl.ds(off, _K)])
        pltpu.sync_copy(bufr, pr_hbm.at[pl.ds(off, _K)])
        return carry

    lax.fori_loop(0, _G, body, 0)


@functools.partial(
    pl.kernel,
    out_type=jax.ShapeDtypeStruct((_NC, _NPAD, _C), jnp.float32),
    mesh=_sc_mesh,
    scratch_types=[
        pltpu.VMEM((_KS,), jnp.int32),
        pltpu.VMEM((_KS,), jnp.int32),
        pltpu.VMEM((_KS,), jnp.int32),
        pltpu.VMEM((_KS,), jnp.int32),
        pltpu.VMEM((_KS, _C), jnp.float32),
        pltpu.VMEM((_KS, _C), jnp.float32),
        pltpu.VMEM((_KS // 8, 8, 128), jnp.float32),
        pltpu.VMEM((_KS // 8, 8, 128), jnp.float32),
        pltpu.VMEM((_RPS, _C), jnp.float32),
        pltpu.VMEM_SHARED((_NPAD, _C), jnp.float32),
        pltpu.SemaphoreType.DMA,
        pltpu.SemaphoreType.DMA,
        pltpu.SemaphoreType.DMA,
        pltpu.SemaphoreType.DMA,
        pltpu.SemaphoreType.DMA,
        pltpu.SemaphoreType.DMA,
    ],
    compiler_params=pltpu.CompilerParams(use_tc_tiling_on_sc=False),
)
def _scatter_messages(tbl_hbm, m_hbm, snd_hbm, rcv_hbm, out_hbm,
                      is0, ir0, is1, ir1, rows0, rows1, mb0, mb1, zbuf, acc,
                      sg0, sg1, sm0, sm1, si0, si1):
    cid = lax.axis_index("c")
    sid = lax.axis_index("s")
    wid = sid * _NC + cid

    zero = jnp.zeros((16,), jnp.float32)

    @plsc.parallel_loop(0, _RPS, unroll=4)
    def _zb(i):
        zbuf[i, pl.ds(0, 16)] = zero
        zbuf[i, pl.ds(16, 16)] = zero

    pltpu.sync_copy(zbuf, acc.at[pl.ds(sid * _RPS, _RPS)])
    plsc.subcore_barrier()

    base = wid * _EPW

    def issue_idx(g, isb, irb, sem):
        off = base + g * _KS
        pltpu.async_copy(snd_hbm.at[pl.ds(off, _KS)], isb, sem)
        pltpu.async_copy(rcv_hbm.at[pl.ds(off, _KS)], irb, sem)

    def drain_idx(isb, irb, sem):
        pltpu.make_async_copy(snd_hbm.at[pl.ds(0, _KS)], isb, sem).wait()
        pltpu.make_async_copy(rcv_hbm.at[pl.ds(0, _KS)], irb, sem).wait()

    def issue_gm(g, isb, rowsb, mbb, semg, semm):
        off = base + g * _KS
        pltpu.async_copy(tbl_hbm.at[isb], rowsb, semg)
        pltpu.async_copy(m_hbm.at[pl.ds(off // 8, _KS // 8)], mbb, semm)

    def drain_gm(rowsb, mbb, semg, semm):
        pltpu.make_async_copy(tbl_hbm.at[pl.ds(0, _KS)], rowsb, semg).wait()
        pltpu.make_async_copy(m_hbm.at[pl.ds(0, _KS // 8)], mbb, semm).wait()

    def mul_scatter(rowsb, mbb, irb):
        @plsc.parallel_loop(0, _KS // 8, unroll=2)
        def _mul(t):
            for s in range(8):
                rr = t * 8 + s
                rowsb[rr, pl.ds(0, 16)] = rowsb[rr, pl.ds(0, 16)] * mbb[t, s, pl.ds(0, 16)]
                rowsb[rr, pl.ds(16, 16)] = rowsb[rr, pl.ds(16, 16)] * mbb[t, s, pl.ds(16, 16)]

        pltpu.sync_copy(rowsb, acc.at[irb], add=True)

    pltpu.sync_copy(snd_hbm.at[pl.ds(base, _KS)], is0)
    pltpu.sync_copy(rcv_hbm.at[pl.ds(base, _KS)], ir0)
    issue_gm(0, is0, rows0, mb0, sg0, sm0)
    issue_idx(1, is1, ir1, si1)

    def body(j, carry):
        not_last = j < _GS // 2 - 1
        drain_gm(rows0, mb0, sg0, sm0)
        drain_idx(is1, ir1, si1)
        issue_gm(2 * j + 1, is1, rows1, mb1, sg1, sm1)
        mul_scatter(rows0, mb0, ir0)

        @pl.when(not_last)
        def _():
            issue_idx(2 * j + 2, is0, ir0, si0)

        drain_gm(rows1, mb1, sg1, sm1)

        @pl.when(not_last)
        def _():
            drain_idx(is0, ir0, si0)
            issue_gm(2 * j + 2, is0, rows0, mb0, sg0, sm0)

        mul_scatter(rows1, mb1, ir1)

        @pl.when(not_last)
        def _():
            issue_idx(2 * j + 3, is1, ir1, si1)

        return carry

    lax.fori_loop(0, _GS // 2, body, 0)
    plsc.subcore_barrier()
    pltpu.sync_copy(acc.at[pl.ds(sid * _RPS, _RPS)],
                    out_hbm.at[cid, pl.ds(sid * _RPS, _RPS)])


def _silu(x):
    return x / (1.0 + jnp.exp(-x))


def _prep_body(na_ref, we_ref, out_ref):
    out_ref[...] = jnp.dot(na_ref[...], we_ref[...],
                           preferred_element_type=jnp.float32)


_T = 6400


def _mlp_body(pst_ref, prt_ref, w1at, w2at, w3a, w1bt, w2bt, w3b,
              ma_ref, mb_ref):
    dt = prt_ref[...] - pst_ref[...]
    q = dt * dt
    ones18 = jnp.ones((1, 8), jnp.float32)
    r2t = jnp.dot(ones18, q, preferred_element_type=jnp.float32) + 1e-12
    rt = jnp.sqrt(r2t)
    x = rt * (1.0 / _R_MAX)
    x2 = x * x
    x4 = x2 * x2
    x5 = x4 * x
    env = 1.0 - 21.0 * x5 + 35.0 * x5 * x - 15.0 * x5 * x2
    env = jnp.where(x < 1.0, env, 0.0)
    scale = env * (jnp.sqrt(2.0 / _R_MAX) / rt)
    n = 1.0 + lax.broadcasted_iota(jnp.int32, (8, 1), 0).astype(jnp.float32)
    arg = n * ((jnp.pi / _R_MAX) * rt)
    ef = jnp.sin(arg) * scale

    ha = _silu(jnp.dot(w1at[...], ef, preferred_element_type=jnp.float32))
    ha = _silu(jnp.dot(w2at[...], ha, preferred_element_type=jnp.float32))
    ma = lax.dot_general(
        ha, w3a[...], (((0,), (0,)), ((), ())),
        preferred_element_type=jnp.float32)
    ma_ref[:, :, pl.ds(0, _C)] = ma.reshape(_T // 8, 8, _C)

    hb = _silu(jnp.dot(w1bt[...], ef, preferred_element_type=jnp.float32))
    hb = _silu(jnp.dot(w2bt[...], hb, preferred_element_type=jnp.float32))
    mb = lax.dot_general(
        hb, w3b[...], (((0,), (0,)), ((), ())),
        preferred_element_type=jnp.float32)
    mb_ref[:, :, pl.ds(0, _C)] = mb.reshape(_T // 8, 8, _C)


def _update_body(agg_ref, na_ref, wel_ref, ns_ref, out_ref):
    a = agg_ref[...]
    agg = a[0, :_N] + a[1, :_N]
    wel = jnp.dot(na_ref[...], wel_ref[...], preferred_element_type=jnp.float32)
    out_ref[...] = agg * _INV_AVG * wel + ns_ref[...]


def _final_body(aggb_ref, s1_ref, na_ref, welb_ref, wra_ref, wrb_ref, ae_ref,
                out_ref):
    a = aggb_ref[...]
    aggb = a[0, :_N] + a[1, :_N]
    na = na_ref[...]
    s1 = s1_ref[...]
    welb = jnp.dot(na, welb_ref[...], preferred_element_type=jnp.float32)
    s2 = aggb * _INV_AVG * welb + s1
    e0 = jnp.dot(na, ae_ref[...], preferred_element_type=jnp.float32)
    e1 = jnp.dot(s1, wra_ref[...], preferred_element_type=jnp.float32)
    e2 = jnp.dot(_silu(s2), wrb_ref[...], preferred_element_type=jnp.float32)
    out_ref[...] = jnp.sum(e0 + e1 + e2).reshape(1, 1)


def kernel(positions, node_attrs, W_embed, atomic_energies,
           Wr1_a, Wr2_a, Wr3_a, Welem_a, Wread_a,
           Wr1_b, Wr2_b, Wr3_b, Welem_b, Wread_b,
           edge_index, batch):
    sender = edge_index[0]
    receiver = edge_index[1]
    pos_pad = jnp.pad(positions, ((0, 0), (0, 5)))
    w3a = Wr3_a[:, :_C]
    w3b = Wr3_b[:, :_C]
    ae = atomic_energies.reshape(-1, 1)

    pos_s, pos_r = _gather_positions(pos_pad, sender, receiver)

    node_scalars = pl.pallas_call(
        _prep_body,
        out_shape=jax.ShapeDtypeStruct((_N, _C), jnp.float32),
    )(node_attrs, W_embed)

    pst = pos_s.T[:8]
    prt = pos_r.T[:8]
    w1at, w2at = Wr1_a.T, Wr2_a.T
    w1bt, w2bt = Wr1_b.T, Wr2_b.T
    nt = _E // _T
    full = lambda shape: pl.BlockSpec(shape, lambda i: (0, 0))
    m_a, m_b = pl.pallas_call(
        _mlp_body,
        grid=(nt,),
        in_specs=[
            pl.BlockSpec((8, _T), lambda i: (0, i)),
            pl.BlockSpec((8, _T), lambda i: (0, i)),
            full(w1at.shape), full(w2at.shape), full(w3a.shape),
            full(w1bt.shape), full(w2bt.shape), full(w3b.shape),
        ],
        out_specs=[
            pl.BlockSpec((_T // 8, 8, 128), lambda i: (i, 0, 0)),
            pl.BlockSpec((_T // 8, 8, 128), lambda i: (i, 0, 0)),
        ],
        out_shape=[
            jax.ShapeDtypeStruct((_E // 8, 8, 128), jnp.float32),
            jax.ShapeDtypeStruct((_E // 8, 8, 128), jnp.float32),
        ],
        compiler_params=pltpu.CompilerParams(
            fuse_transposed_lhs_in_matmul=True),
    )(pst, prt, w1at, w2at, w3a, w1bt, w2bt, w3b)

    agg_a = _scatter_messages(node_scalars, m_a, sender, receiver)

    s1 = pl.pallas_call(
        _update_body,
        out_shape=jax.ShapeDtypeStruct((_N, _C), jnp.float32),
    )(agg_a, node_attrs, Welem_a, node_scalars)

    agg_b = _scatter_messages(s1, m_b, sender, receiver)

    e = pl.pallas_call(
        _final_body,
        out_shape=jax.ShapeDtypeStruct((1, 1), jnp.float32),
    )(agg_b, s1, node_attrs, Welem_b, Wread_a, Wread_b, ae)

    return e[0]


# speedup vs baseline: 2.0006x; 1.5404x over previous
import functools

import jax
import jax.numpy as jnp
from jax import lax
from jax.experimental import pallas as pl
from jax.experimental.pallas import tpu as pltpu
from jax.experimental.pallas import tpu_sc as plsc

_N = 10000
_E = 320000
_C = 32
_R_MAX = 5.0
_INV_AVG = 1.0 / 32.0

_NC = 2
_NS = 16
_NW = _NC * _NS
_EPW = _E // _NW
_K = 1000
_G = _EPW // _K
_KS = 200
_GS = _EPW // _KS
_NPAD = 10240
_RPS = _NPAD // _NS

_sc_mesh = plsc.VectorSubcoreMesh(core_axis_name="c", subcore_axis_name="s")


@functools.partial(
    pl.kernel,
    out_type=jax.ShapeDtypeStruct((_E,), jnp.float32),
    mesh=_sc_mesh,
    scratch_types=[
        pltpu.VMEM((_N, 4), jnp.float32),
        pltpu.VMEM((_K,), jnp.int32),
        pltpu.VMEM((_K,), jnp.int32),
        pltpu.VMEM((_K,), jnp.float32),
    ],
    compiler_params=pltpu.CompilerParams(use_tc_tiling_on_sc=False,
                                         needs_layout_passes=False),
)
def _edge_r2(pos_hbm, snd_hbm, rcv_hbm, r2_hbm, ptbl, isb, irb, r2b):
    wid = lax.axis_index("s") * _NC + lax.axis_index("c")
    pltpu.sync_copy(pos_hbm, ptbl)
    base = wid * _EPW

    def body(g, carry):
        off = base + g * _K
        pltpu.sync_copy(snd_hbm.at[pl.ds(off, _K)], isb)
        pltpu.sync_copy(rcv_hbm.at[pl.ds(off, _K)], irb)

        @plsc.parallel_loop(0, _K // 16, unroll=2)
        def _grp(t):
            sidx = isb[pl.ds(t * 16, 16)]
            ridx = irb[pl.ds(t * 16, 16)]
            r2 = None
            for c in range(3):
                cc = jnp.full((16,), c, jnp.int32)
                d = (plsc.load_gather(ptbl, [ridx, cc])
                     - plsc.load_gather(ptbl, [sidx, cc]))
                r2 = d * d if r2 is None else r2 + d * d
            r2b[pl.ds(t * 16, 16)] = r2

        pltpu.sync_copy(r2b, r2_hbm.at[pl.ds(off, _K)])
        return carry

    lax.fori_loop(0, _G, body, 0)


@functools.partial(
    pl.kernel,
    out_type=jax.ShapeDtypeStruct((_NC, _NPAD, _C), jnp.float32),
    mesh=_sc_mesh,
    scratch_types=[
        pltpu.VMEM((_KS,), jnp.int32),
        pltpu.VMEM((_KS,), jnp.int32),
        pltpu.VMEM((_KS,), jnp.int32),
        pltpu.VMEM((_KS,), jnp.int32),
        pltpu.VMEM((_KS, _C), jnp.float32),
        pltpu.VMEM((_KS, _C), jnp.float32),
        pltpu.VMEM((_KS // 8, 8, 128), jnp.float32),
        pltpu.VMEM((_KS // 8, 8, 128), jnp.float32),
        pltpu.VMEM((_RPS, _C), jnp.float32),
        pltpu.VMEM_SHARED((_NPAD, _C), jnp.float32),
        pltpu.SemaphoreType.DMA,
        pltpu.SemaphoreType.DMA,
        pltpu.SemaphoreType.DMA,
        pltpu.SemaphoreType.DMA,
        pltpu.SemaphoreType.DMA,
        pltpu.SemaphoreType.DMA,
    ],
    compiler_params=pltpu.CompilerParams(use_tc_tiling_on_sc=False),
)
def _scatter_messages(tbl_hbm, m_hbm, snd_hbm, rcv_hbm, out_hbm,
                      is0, ir0, is1, ir1, rows0, rows1, mb0, mb1, zbuf, acc,
                      sg0, sg1, sm0, sm1, si0, si1):
    cid = lax.axis_index("c")
    sid = lax.axis_index("s")
    wid = sid * _NC + cid

    zero = jnp.zeros((16,), jnp.float32)

    @plsc.parallel_loop(0, _RPS, unroll=4)
    def _zb(i):
        zbuf[i, pl.ds(0, 16)] = zero
        zbuf[i, pl.ds(16, 16)] = zero

    pltpu.sync_copy(zbuf, acc.at[pl.ds(sid * _RPS, _RPS)])
    plsc.subcore_barrier()

    base = wid * _EPW

    def issue_idx(g, isb, irb, sem):
        off = base + g * _KS
        pltpu.async_copy(snd_hbm.at[pl.ds(off, _KS)], isb, sem)
        pltpu.async_copy(rcv_hbm.at[pl.ds(off, _KS)], irb, sem)

    def drain_idx(isb, irb, sem):
        pltpu.make_async_copy(snd_hbm.at[pl.ds(0, _KS)], isb, sem).wait()
        pltpu.make_async_copy(rcv_hbm.at[pl.ds(0, _KS)], irb, sem).wait()

    def issue_gm(g, isb, rowsb, mbb, semg, semm):
        off = base + g * _KS
        pltpu.async_copy(tbl_hbm.at[isb], rowsb, semg)
        pltpu.async_copy(m_hbm.at[pl.ds(off // 8, _KS // 8)], mbb, semm)

    def drain_gm(rowsb, mbb, semg, semm):
        pltpu.make_async_copy(tbl_hbm.at[pl.ds(0, _KS)], rowsb, semg).wait()
        pltpu.make_async_copy(m_hbm.at[pl.ds(0, _KS // 8)], mbb, semm).wait()

    def mul_scatter(rowsb, mbb, irb):
        @plsc.parallel_loop(0, _KS // 8, unroll=2)
        def _mul(t):
            for s in range(8):
                rr = t * 8 + s
                rowsb[rr, pl.ds(0, 16)] = rowsb[rr, pl.ds(0, 16)] * mbb[t, s, pl.ds(0, 16)]
                rowsb[rr, pl.ds(16, 16)] = rowsb[rr, pl.ds(16, 16)] * mbb[t, s, pl.ds(16, 16)]

        pltpu.sync_copy(rowsb, acc.at[irb], add=True)

    pltpu.sync_copy(snd_hbm.at[pl.ds(base, _KS)], is0)
    pltpu.sync_copy(rcv_hbm.at[pl.ds(base, _KS)], ir0)
    issue_gm(0, is0, rows0, mb0, sg0, sm0)
    issue_idx(1, is1, ir1, si1)

    def body(j, carry):
        not_last = j < _GS // 2 - 1
        drain_gm(rows0, mb0, sg0, sm0)
        drain_idx(is1, ir1, si1)
        issue_gm(2 * j + 1, is1, rows1, mb1, sg1, sm1)
        mul_scatter(rows0, mb0, ir0)

        @pl.when(not_last)
        def _():
            issue_idx(2 * j + 2, is0, ir0, si0)

        drain_gm(rows1, mb1, sg1, sm1)

        @pl.when(not_last)
        def _():
            drain_idx(is0, ir0, si0)
            issue_gm(2 * j + 2, is0, rows0, mb0, sg0, sm0)

        mul_scatter(rows1, mb1, ir1)

        @pl.when(not_last)
        def _():
            issue_idx(2 * j + 3, is1, ir1, si1)

        return carry

    lax.fori_loop(0, _GS // 2, body, 0)
    plsc.subcore_barrier()
    pltpu.sync_copy(acc.at[pl.ds(sid * _RPS, _RPS)],
                    out_hbm.at[cid, pl.ds(sid * _RPS, _RPS)])


def _silu(x):
    return x / (1.0 + jnp.exp(-x))


def _prep_body(na_ref, we_ref, out_ref):
    out_ref[...] = jnp.dot(na_ref[...], we_ref[...],
                           preferred_element_type=jnp.float32)


_T = 6400


def _mlp_body(r2_ref, w1at, w2at, w3a, w1bt, w2bt, w3b, ma_ref, mb_ref):
    r2t = r2_ref[...] + 1e-12
    rt = jnp.sqrt(r2t)
    x = rt * (1.0 / _R_MAX)
    x2 = x * x
    x4 = x2 * x2
    x5 = x4 * x
    env = 1.0 - 21.0 * x5 + 35.0 * x5 * x - 15.0 * x5 * x2
    env = jnp.where(x < 1.0, env, 0.0)
    scale = env * (jnp.sqrt(2.0 / _R_MAX) / rt)
    n = 1.0 + lax.broadcasted_iota(jnp.int32, (8, 1), 0).astype(jnp.float32)
    arg = n * ((jnp.pi / _R_MAX) * rt)
    ef = jnp.sin(arg) * scale

    ha = _silu(jnp.dot(w1at[...], ef, preferred_element_type=jnp.float32))
    ha = _silu(jnp.dot(w2at[...], ha, preferred_element_type=jnp.float32))
    ma = lax.dot_general(
        ha, w3a[...], (((0,), (0,)), ((), ())),
        preferred_element_type=jnp.float32)
    ma_ref[:, :, pl.ds(0, _C)] = ma.reshape(_T // 8, 8, _C)

    hb = _silu(jnp.dot(w1bt[...], ef, preferred_element_type=jnp.float32))
    hb = _silu(jnp.dot(w2bt[...], hb, preferred_element_type=jnp.float32))
    mb = lax.dot_general(
        hb, w3b[...], (((0,), (0,)), ((), ())),
        preferred_element_type=jnp.float32)
    mb_ref[:, :, pl.ds(0, _C)] = mb.reshape(_T // 8, 8, _C)


def _update_body(agg_ref, na_ref, wel_ref, ns_ref, out_ref):
    a = agg_ref[...]
    agg = a[0, :_N] + a[1, :_N]
    wel = jnp.dot(na_ref[...], wel_ref[...], preferred_element_type=jnp.float32)
    out_ref[...] = agg * _INV_AVG * wel + ns_ref[...]


def _final_body(aggb_ref, s1_ref, na_ref, welb_ref, wra_ref, wrb_ref, ae_ref,
                out_ref):
    a = aggb_ref[...]
    aggb = a[0, :_N] + a[1, :_N]
    na = na_ref[...]
    s1 = s1_ref[...]
    welb = jnp.dot(na, welb_ref[...], preferred_element_type=jnp.float32)
    s2 = aggb * _INV_AVG * welb + s1
    e0 = jnp.dot(na, ae_ref[...], preferred_element_type=jnp.float32)
    e1 = jnp.dot(s1, wra_ref[...], preferred_element_type=jnp.float32)
    e2 = jnp.dot(_silu(s2), wrb_ref[...], preferred_element_type=jnp.float32)
    out_ref[...] = jnp.sum(e0 + e1 + e2).reshape(1, 1)


def kernel(positions, node_attrs, W_embed, atomic_energies,
           Wr1_a, Wr2_a, Wr3_a, Welem_a, Wread_a,
           Wr1_b, Wr2_b, Wr3_b, Welem_b, Wread_b,
           edge_index, batch):
    sender = edge_index[0]
    receiver = edge_index[1]
    pos_pad = jnp.pad(positions, ((0, 0), (0, 1)))
    w3a = Wr3_a[:, :_C]
    w3b = Wr3_b[:, :_C]
    ae = atomic_energies.reshape(-1, 1)

    r2 = _edge_r2(pos_pad, sender, receiver)
    r2row = r2.reshape(1, _E)

    node_scalars = pl.pallas_call(
        _prep_body,
        out_shape=jax.ShapeDtypeStruct((_N, _C), jnp.float32),
    )(node_attrs, W_embed)

    w1at, w2at = Wr1_a.T, Wr2_a.T
    w1bt, w2bt = Wr1_b.T, Wr2_b.T
    nt = _E // _T
    full = lambda shape: pl.BlockSpec(shape, lambda i: (0, 0))
    m_a, m_b = pl.pallas_call(
        _mlp_body,
        grid=(nt,),
        in_specs=[
            pl.BlockSpec((1, _T), lambda i: (0, i)),
            full(w1at.shape), full(w2at.shape), full(w3a.shape),
            full(w1bt.shape), full(w2bt.shape), full(w3b.shape),
        ],
        out_specs=[
            pl.BlockSpec((_T // 8, 8, 128), lambda i: (i, 0, 0)),
            pl.BlockSpec((_T // 8, 8, 128), lambda i: (i, 0, 0)),
        ],
        out_shape=[
            jax.ShapeDtypeStruct((_E // 8, 8, 128), jnp.float32),
            jax.ShapeDtypeStruct((_E // 8, 8, 128), jnp.float32),
        ],
        compiler_params=pltpu.CompilerParams(
            fuse_transposed_lhs_in_matmul=True),
    )(r2row, w1at, w2at, w3a, w1bt, w2bt, w3b)

    agg_a = _scatter_messages(node_scalars, m_a, sender, receiver)

    s1 = pl.pallas_call(
        _update_body,
        out_shape=jax.ShapeDtypeStruct((_N, _C), jnp.float32),
    )(agg_a, node_attrs, Welem_a, node_scalars)

    agg_b = _scatter_messages(s1, m_b, sender, receiver)

    e = pl.pallas_call(
        _final_body,
        out_shape=jax.ShapeDtypeStruct((1, 1), jnp.float32),
    )(agg_b, s1, node_attrs, Welem_b, Wread_a, Wread_b, ae)

    return e[0]


# speedup vs baseline: 2.2914x; 1.1453x over previous
import functools

import jax
import jax.numpy as jnp
from jax import lax
from jax.experimental import pallas as pl
from jax.experimental.pallas import tpu as pltpu
from jax.experimental.pallas import tpu_sc as plsc

_N = 10000
_E = 320000
_C = 32
_R_MAX = 5.0
_INV_AVG = 1.0 / 32.0

_NC = 2
_NS = 16
_NW = _NC * _NS
_EPW = _E // _NW
_K = 1000
_G = _EPW // _K
_KS = 200
_GS = _EPW // _KS
_NPAD = 10240
_RPS = _NPAD // _NS

_sc_mesh = plsc.VectorSubcoreMesh(core_axis_name="c", subcore_axis_name="s")


@functools.partial(
    pl.kernel,
    out_type=jax.ShapeDtypeStruct((_E,), jnp.float32),
    mesh=_sc_mesh,
    scratch_types=[
        pltpu.VMEM((_N, 4), jnp.float32),
        pltpu.VMEM((_K,), jnp.int32),
        pltpu.VMEM((_K,), jnp.int32),
        pltpu.VMEM((_K,), jnp.float32),
    ],
    compiler_params=pltpu.CompilerParams(use_tc_tiling_on_sc=False,
                                         needs_layout_passes=False),
)
def _edge_r2(pos_hbm, snd_hbm, rcv_hbm, r2_hbm, ptbl, isb, irb, r2b):
    wid = lax.axis_index("s") * _NC + lax.axis_index("c")
    pltpu.sync_copy(pos_hbm, ptbl)
    base = wid * _EPW

    def body(g, carry):
        off = base + g * _K
        pltpu.sync_copy(snd_hbm.at[pl.ds(off, _K)], isb)
        pltpu.sync_copy(rcv_hbm.at[pl.ds(off, _K)], irb)

        @plsc.parallel_loop(0, _K // 16, unroll=2)
        def _grp(t):
            sidx = isb[pl.ds(t * 16, 16)]
            ridx = irb[pl.ds(t * 16, 16)]
            r2 = None
            for c in range(3):
                cc = jnp.full((16,), c, jnp.int32)
                d = (plsc.load_gather(ptbl, [ridx, cc])
                     - plsc.load_gather(ptbl, [sidx, cc]))
                r2 = d * d if r2 is None else r2 + d * d
            r2b[pl.ds(t * 16, 16)] = r2

        pltpu.sync_copy(r2b, r2_hbm.at[pl.ds(off, _K)])
        return carry

    lax.fori_loop(0, _G, body, 0)


@functools.partial(
    pl.kernel,
    out_type=jax.ShapeDtypeStruct((_NC, _NPAD, _C), jnp.float32),
    mesh=_sc_mesh,
    scratch_types=[
        pltpu.VMEM((_KS,), jnp.int32),
        pltpu.VMEM((_KS,), jnp.int32),
        pltpu.VMEM((_KS,), jnp.int32),
        pltpu.VMEM((_KS,), jnp.int32),
        pltpu.VMEM((_KS, _C), jnp.float32),
        pltpu.VMEM((_KS, _C), jnp.float32),
        pltpu.VMEM((_KS // 8, 8, _C), jnp.float32),
        pltpu.VMEM((_KS // 8, 8, _C), jnp.float32),
        pltpu.VMEM((_RPS, _C), jnp.float32),
        pltpu.VMEM_SHARED((_NPAD, _C), jnp.float32),
        pltpu.SemaphoreType.DMA,
        pltpu.SemaphoreType.DMA,
        pltpu.SemaphoreType.DMA,
        pltpu.SemaphoreType.DMA,
        pltpu.SemaphoreType.DMA,
        pltpu.SemaphoreType.DMA,
    ],
    compiler_params=pltpu.CompilerParams(use_tc_tiling_on_sc=False),
)
def _scatter_messages(tbl_hbm, m_hbm, snd_hbm, rcv_hbm, out_hbm,
                      is0, ir0, is1, ir1, rows0, rows1, mb0, mb1, zbuf, acc,
                      sg0, sg1, sm0, sm1, si0, si1):
    cid = lax.axis_index("c")
    sid = lax.axis_index("s")
    wid = sid * _NC + cid

    zero = jnp.zeros((16,), jnp.float32)

    @plsc.parallel_loop(0, _RPS, unroll=4)
    def _zb(i):
        zbuf[i, pl.ds(0, 16)] = zero
        zbuf[i, pl.ds(16, 16)] = zero

    pltpu.sync_copy(zbuf, acc.at[pl.ds(sid * _RPS, _RPS)])
    plsc.subcore_barrier()

    base = wid * _EPW

    def issue_idx(g, isb, irb, sem):
        off = base + g * _KS
        pltpu.async_copy(snd_hbm.at[pl.ds(off, _KS)], isb, sem)
        pltpu.async_copy(rcv_hbm.at[pl.ds(off, _KS)], irb, sem)

    def drain_idx(isb, irb, sem):
        pltpu.make_async_copy(snd_hbm.at[pl.ds(0, _KS)], isb, sem).wait()
        pltpu.make_async_copy(rcv_hbm.at[pl.ds(0, _KS)], irb, sem).wait()

    def issue_gm(g, isb, rowsb, mbb, semg, semm):
        off = base + g * _KS
        pltpu.async_copy(tbl_hbm.at[isb], rowsb, semg)
        pltpu.async_copy(
            m_hbm.at[pl.ds(off // 8, _KS // 8), :, pl.ds(0, _C)], mbb, semm)

    def drain_gm(rowsb, mbb, semg, semm):
        pltpu.make_async_copy(tbl_hbm.at[pl.ds(0, _KS)], rowsb, semg).wait()
        pltpu.make_async_copy(
            m_hbm.at[pl.ds(0, _KS // 8), :, pl.ds(0, _C)], mbb, semm).wait()

    def mul_scatter(rowsb, mbb, irb):
        @plsc.parallel_loop(0, _KS // 8, unroll=2)
        def _mul(t):
            for s in range(8):
                rr = t * 8 + s
                rowsb[rr, pl.ds(0, 16)] = rowsb[rr, pl.ds(0, 16)] * mbb[t, s, pl.ds(0, 16)]
                rowsb[rr, pl.ds(16, 16)] = rowsb[rr, pl.ds(16, 16)] * mbb[t, s, pl.ds(16, 16)]

        pltpu.sync_copy(rowsb, acc.at[irb], add=True)

    pltpu.sync_copy(snd_hbm.at[pl.ds(base, _KS)], is0)
    pltpu.sync_copy(rcv_hbm.at[pl.ds(base, _KS)], ir0)
    issue_gm(0, is0, rows0, mb0, sg0, sm0)
    issue_idx(1, is1, ir1, si1)

    def body(j, carry):
        not_last = j < _GS // 2 - 1
        drain_gm(rows0, mb0, sg0, sm0)
        drain_idx(is1, ir1, si1)
        issue_gm(2 * j + 1, is1, rows1, mb1, sg1, sm1)
        mul_scatter(rows0, mb0, ir0)

        @pl.when(not_last)
        def _():
            issue_idx(2 * j + 2, is0, ir0, si0)

        drain_gm(rows1, mb1, sg1, sm1)

        @pl.when(not_last)
        def _():
            drain_idx(is0, ir0, si0)
            issue_gm(2 * j + 2, is0, rows0, mb0, sg0, sm0)

        mul_scatter(rows1, mb1, ir1)

        @pl.when(not_last)
        def _():
            issue_idx(2 * j + 3, is1, ir1, si1)

        return carry

    lax.fori_loop(0, _GS // 2, body, 0)
    plsc.subcore_barrier()
    pltpu.sync_copy(acc.at[pl.ds(sid * _RPS, _RPS)],
                    out_hbm.at[cid, pl.ds(sid * _RPS, _RPS)])


def _silu(x):
    return x / (1.0 + jnp.exp(-x))


def _prep_body(na_ref, we_ref, out_ref):
    out_ref[...] = jnp.dot(na_ref[...], we_ref[...],
                           preferred_element_type=jnp.float32)


_T = 6400


def _mlp_body(r2_ref, w1at, w2at, w3a, w1bt, w2bt, w3b, ma_ref, mb_ref):
    r2t = r2_ref[...] + 1e-12
    rt = jnp.sqrt(r2t)
    x = rt * (1.0 / _R_MAX)
    x2 = x * x
    x4 = x2 * x2
    x5 = x4 * x
    env = 1.0 - 21.0 * x5 + 35.0 * x5 * x - 15.0 * x5 * x2
    env = jnp.where(x < 1.0, env, 0.0)
    scale = env * (jnp.sqrt(2.0 / _R_MAX) / rt)
    n = 1.0 + lax.broadcasted_iota(jnp.int32, (8, 1), 0).astype(jnp.float32)
    arg = n * ((jnp.pi / _R_MAX) * rt)
    ef = jnp.sin(arg) * scale

    ha = _silu(jnp.dot(w1at[...], ef, preferred_element_type=jnp.float32))
    ha = _silu(jnp.dot(w2at[...], ha, preferred_element_type=jnp.float32))
    ma = lax.dot_general(
        ha, w3a[...], (((0,), (0,)), ((), ())),
        preferred_element_type=jnp.float32)
    ma_ref[:, :, pl.ds(0, _C)] = ma.reshape(_T // 8, 8, _C)

    hb = _silu(jnp.dot(w1bt[...], ef, preferred_element_type=jnp.float32))
    hb = _silu(jnp.dot(w2bt[...], hb, preferred_element_type=jnp.float32))
    mb = lax.dot_general(
        hb, w3b[...], (((0,), (0,)), ((), ())),
        preferred_element_type=jnp.float32)
    mb_ref[:, :, pl.ds(0, _C)] = mb.reshape(_T // 8, 8, _C)


def _update_body(agg_ref, na_ref, wel_ref, ns_ref, out_ref):
    a = agg_ref[...]
    agg = a[0, :_N] + a[1, :_N]
    wel = jnp.dot(na_ref[...], wel_ref[...], preferred_element_type=jnp.float32)
    out_ref[...] = agg * _INV_AVG * wel + ns_ref[...]


def _final_body(aggb_ref, s1_ref, na_ref, welb_ref, wra_ref, wrb_ref, ae_ref,
                out_ref):
    a = aggb_ref[...]
    aggb = a[0, :_N] + a[1, :_N]
    na = na_ref[...]
    s1 = s1_ref[...]
    welb = jnp.dot(na, welb_ref[...], preferred_element_type=jnp.float32)
    s2 = aggb * _INV_AVG * welb + s1
    e0 = jnp.dot(na, ae_ref[...], preferred_element_type=jnp.float32)
    e1 = jnp.dot(s1, wra_ref[...], preferred_element_type=jnp.float32)
    e2 = jnp.dot(_silu(s2), wrb_ref[...], preferred_element_type=jnp.float32)
    out_ref[...] = jnp.sum(e0 + e1 + e2).reshape(1, 1)


def kernel(positions, node_attrs, W_embed, atomic_energies,
           Wr1_a, Wr2_a, Wr3_a, Welem_a, Wread_a,
           Wr1_b, Wr2_b, Wr3_b, Welem_b, Wread_b,
           edge_index, batch):
    sender = edge_index[0]
    receiver = edge_index[1]
    pos_pad = jnp.pad(positions, ((0, 0), (0, 1)))
    w3a = Wr3_a[:, :_C]
    w3b = Wr3_b[:, :_C]
    ae = atomic_energies.reshape(-1, 1)

    r2 = _edge_r2(pos_pad, sender, receiver)
    r2row = r2.reshape(1, _E)

    node_scalars = pl.pallas_call(
        _prep_body,
        out_shape=jax.ShapeDtypeStruct((_N, _C), jnp.float32),
    )(node_attrs, W_embed)

    w1at, w2at = Wr1_a.T, Wr2_a.T
    w1bt, w2bt = Wr1_b.T, Wr2_b.T
    nt = _E // _T
    full = lambda shape: pl.BlockSpec(shape, lambda i: (0, 0))
    m_a, m_b = pl.pallas_call(
        _mlp_body,
        grid=(nt,),
        in_specs=[
            pl.BlockSpec((1, _T), lambda i: (0, i)),
            full(w1at.shape), full(w2at.shape), full(w3a.shape),
            full(w1bt.shape), full(w2bt.shape), full(w3b.shape),
        ],
        out_specs=[
            pl.BlockSpec((_T // 8, 8, 128), lambda i: (i, 0, 0)),
            pl.BlockSpec((_T // 8, 8, 128), lambda i: (i, 0, 0)),
        ],
        out_shape=[
            jax.ShapeDtypeStruct((_E // 8, 8, 128), jnp.float32),
            jax.ShapeDtypeStruct((_E // 8, 8, 128), jnp.float32),
        ],
        compiler_params=pltpu.CompilerParams(
            fuse_transposed_lhs_in_matmul=True),
    )(r2row, w1at, w2at, w3a, w1bt, w2bt, w3b)

    agg_a = _scatter_messages(node_scalars, m_a, sender, receiver)

    s1 = pl.pallas_call(
        _update_body,
        out_shape=jax.ShapeDtypeStruct((_N, _C), jnp.float32),
    )(agg_a, node_attrs, Welem_a, node_scalars)

    agg_b = _scatter_messages(s1, m_b, sender, receiver)

    e = pl.pallas_call(
        _final_body,
        out_shape=jax.ShapeDtypeStruct((1, 1), jnp.float32),
    )(agg_b, s1, node_attrs, Welem_b, Wread_a, Wread_b, ae)

    return e[0]


# speedup vs baseline: 2.3653x; 1.0322x over previous
import functools

import jax
import jax.numpy as jnp
from jax import lax
from jax.experimental import pallas as pl
from jax.experimental.pallas import tpu as pltpu
from jax.experimental.pallas import tpu_sc as plsc

_N = 10000
_E = 320000
_C = 32
_R_MAX = 5.0
_INV_AVG = 1.0 / 32.0

_NC = 2
_NS = 16
_NW = _NC * _NS
_EPW = _E // _NW
_K = 1000
_G = _EPW // _K
_KS = 200
_GS = _EPW // _KS
_NPAD = 10240
_RPS = _NPAD // _NS

_sc_mesh = plsc.VectorSubcoreMesh(core_axis_name="c", subcore_axis_name="s")


@functools.partial(
    pl.kernel,
    out_type=jax.ShapeDtypeStruct((_E,), jnp.float32),
    mesh=_sc_mesh,
    scratch_types=[
        pltpu.VMEM((_N, 4), jnp.float32),
        pltpu.VMEM((_K,), jnp.int32),
        pltpu.VMEM((_K,), jnp.int32),
        pltpu.VMEM((_K,), jnp.float32),
    ],
    compiler_params=pltpu.CompilerParams(use_tc_tiling_on_sc=False,
                                         needs_layout_passes=False),
)
def _edge_r2(pos_hbm, snd_hbm, rcv_hbm, r2_hbm, ptbl, isb, irb, r2b):
    wid = lax.axis_index("s") * _NC + lax.axis_index("c")
    pltpu.sync_copy(pos_hbm, ptbl)
    base = wid * _EPW

    def body(g, carry):
        off = base + g * _K
        pltpu.sync_copy(snd_hbm.at[pl.ds(off, _K)], isb)
        pltpu.sync_copy(rcv_hbm.at[pl.ds(off, _K)], irb)

        @plsc.parallel_loop(0, _K // 16, unroll=2)
        def _grp(t):
            sidx = isb[pl.ds(t * 16, 16)]
            ridx = irb[pl.ds(t * 16, 16)]
            r2 = None
            for c in range(3):
                cc = jnp.full((16,), c, jnp.int32)
                d = (plsc.load_gather(ptbl, [ridx, cc])
                     - plsc.load_gather(ptbl, [sidx, cc]))
                r2 = d * d if r2 is None else r2 + d * d
            r2b[pl.ds(t * 16, 16)] = r2

        pltpu.sync_copy(r2b, r2_hbm.at[pl.ds(off, _K)])
        return carry

    lax.fori_loop(0, _G, body, 0)


@functools.partial(
    pl.kernel,
    out_type=jax.ShapeDtypeStruct((_NC, _NPAD, _C), jnp.float32),
    mesh=_sc_mesh,
    scratch_types=[
        pltpu.VMEM((_KS,), jnp.int32),
        pltpu.VMEM((_KS,), jnp.int32),
        pltpu.VMEM((_KS,), jnp.int32),
        pltpu.VMEM((_KS,), jnp.int32),
        pltpu.VMEM((_KS, _C), jnp.float32),
        pltpu.VMEM((_KS, _C), jnp.float32),
        pltpu.VMEM((_KS // 8, 8, _C), jnp.float32),
        pltpu.VMEM((_KS // 8, 8, _C), jnp.float32),
        pltpu.VMEM((_RPS, _C), jnp.float32),
        pltpu.VMEM_SHARED((_NPAD, _C), jnp.float32),
        pltpu.SemaphoreType.DMA,
        pltpu.SemaphoreType.DMA,
        pltpu.SemaphoreType.DMA,
        pltpu.SemaphoreType.DMA,
        pltpu.SemaphoreType.DMA,
        pltpu.SemaphoreType.DMA,
    ],
    compiler_params=pltpu.CompilerParams(use_tc_tiling_on_sc=False),
)
def _scatter_messages(tbl_hbm, m_hbm, snd_hbm, rcv_hbm, out_hbm,
                      is0, ir0, is1, ir1, rows0, rows1, mb0, mb1, zbuf, acc,
                      sg0, sg1, sm0, sm1, si0, si1):
    cid = lax.axis_index("c")
    sid = lax.axis_index("s")
    wid = sid * _NC + cid

    zero = jnp.zeros((16,), jnp.float32)

    @plsc.parallel_loop(0, _RPS, unroll=4)
    def _zb(i):
        zbuf[i, pl.ds(0, 16)] = zero
        zbuf[i, pl.ds(16, 16)] = zero

    pltpu.sync_copy(zbuf, acc.at[pl.ds(sid * _RPS, _RPS)])
    plsc.subcore_barrier()

    base = wid * _EPW

    def issue_idx(g, isb, irb, sem):
        off = base + g * _KS
        pltpu.async_copy(snd_hbm.at[pl.ds(off, _KS)], isb, sem)
        pltpu.async_copy(rcv_hbm.at[pl.ds(off, _KS)], irb, sem)

    def drain_idx(isb, irb, sem):
        pltpu.make_async_copy(snd_hbm.at[pl.ds(0, _KS)], isb, sem).wait()
        pltpu.make_async_copy(rcv_hbm.at[pl.ds(0, _KS)], irb, sem).wait()

    def issue_gm(g, isb, rowsb, mbb, semg, semm):
        off = base + g * _KS
        pltpu.async_copy(tbl_hbm.at[isb], rowsb, semg)
        pltpu.async_copy(
            m_hbm.at[pl.ds(off // 8, _KS // 8), :, pl.ds(0, _C)], mbb, semm)

    def drain_gm(rowsb, mbb, semg, semm):
        pltpu.make_async_copy(tbl_hbm.at[pl.ds(0, _KS)], rowsb, semg).wait()
        pltpu.make_async_copy(
            m_hbm.at[pl.ds(0, _KS // 8), :, pl.ds(0, _C)], mbb, semm).wait()

    def mul_scatter(rowsb, mbb, irb):
        @plsc.parallel_loop(0, _KS // 8, unroll=2)
        def _mul(t):
            for s in range(8):
                rr = t * 8 + s
                rowsb[rr, pl.ds(0, 16)] = rowsb[rr, pl.ds(0, 16)] * mbb[t, s, pl.ds(0, 16)]
                rowsb[rr, pl.ds(16, 16)] = rowsb[rr, pl.ds(16, 16)] * mbb[t, s, pl.ds(16, 16)]

        pltpu.sync_copy(rowsb, acc.at[irb], add=True)

    pltpu.sync_copy(snd_hbm.at[pl.ds(base, _KS)], is0)
    pltpu.sync_copy(rcv_hbm.at[pl.ds(base, _KS)], ir0)
    issue_gm(0, is0, rows0, mb0, sg0, sm0)
    issue_idx(1, is1, ir1, si1)

    def body(j, carry):
        not_last = j < _GS // 2 - 1
        drain_gm(rows0, mb0, sg0, sm0)
        drain_idx(is1, ir1, si1)
        issue_gm(2 * j + 1, is1, rows1, mb1, sg1, sm1)
        mul_scatter(rows0, mb0, ir0)

        @pl.when(not_last)
        def _():
            issue_idx(2 * j + 2, is0, ir0, si0)

        drain_gm(rows1, mb1, sg1, sm1)

        @pl.when(not_last)
        def _():
            drain_idx(is0, ir0, si0)
            issue_gm(2 * j + 2, is0, rows0, mb0, sg0, sm0)

        mul_scatter(rows1, mb1, ir1)

        @pl.when(not_last)
        def _():
            issue_idx(2 * j + 3, is1, ir1, si1)

        return carry

    lax.fori_loop(0, _GS // 2, body, 0)
    plsc.subcore_barrier()
    pltpu.sync_copy(acc.at[pl.ds(sid * _RPS, _RPS)],
                    out_hbm.at[cid, pl.ds(sid * _RPS, _RPS)])


def _silu(x):
    return x / (1.0 + jnp.exp(-x))


def _prep_body(na_ref, we_ref, out_ref):
    out_ref[...] = jnp.dot(na_ref[...], we_ref[...],
                           preferred_element_type=jnp.float32)


_T = 6400


def _mlp_body(r2_ref, w1t, w2t, w3, m_ref):
    r2t = r2_ref[...] + 1e-12
    rt = jnp.sqrt(r2t)
    x = rt * (1.0 / _R_MAX)
    x2 = x * x
    x4 = x2 * x2
    x5 = x4 * x
    env = 1.0 - 21.0 * x5 + 35.0 * x5 * x - 15.0 * x5 * x2
    env = jnp.where(x < 1.0, env, 0.0)
    scale = env * (jnp.sqrt(2.0 / _R_MAX) / rt)
    n = 1.0 + lax.broadcasted_iota(jnp.int32, (8, 1), 0).astype(jnp.float32)
    arg = n * ((jnp.pi / _R_MAX) * rt)
    ef = jnp.sin(arg) * scale

    h = _silu(jnp.dot(w1t[...], ef, preferred_element_type=jnp.float32))
    h = _silu(jnp.dot(w2t[...], h, preferred_element_type=jnp.float32))
    m = lax.dot_general(
        h, w3[...], (((0,), (0,)), ((), ())),
        preferred_element_type=jnp.float32)
    m_ref[:, :, pl.ds(0, _C)] = m.reshape(_T // 8, 8, _C)


def _update_body(agg_ref, na_ref, wel_ref, ns_ref, out_ref):
    a = agg_ref[...]
    agg = a[0, :_N] + a[1, :_N]
    wel = jnp.dot(na_ref[...], wel_ref[...], preferred_element_type=jnp.float32)
    out_ref[...] = agg * _INV_AVG * wel + ns_ref[...]


def _final_body(aggb_ref, s1_ref, na_ref, welb_ref, wra_ref, wrb_ref, ae_ref,
                out_ref):
    a = aggb_ref[...]
    aggb = a[0, :_N] + a[1, :_N]
    na = na_ref[...]
    s1 = s1_ref[...]
    welb = jnp.dot(na, welb_ref[...], preferred_element_type=jnp.float32)
    s2 = aggb * _INV_AVG * welb + s1
    e0 = jnp.dot(na, ae_ref[...], preferred_element_type=jnp.float32)
    e1 = jnp.dot(s1, wra_ref[...], preferred_element_type=jnp.float32)
    e2 = jnp.dot(_silu(s2), wrb_ref[...], preferred_element_type=jnp.float32)
    out_ref[...] = jnp.sum(e0 + e1 + e2).reshape(1, 1)


def kernel(positions, node_attrs, W_embed, atomic_energies,
           Wr1_a, Wr2_a, Wr3_a, Welem_a, Wread_a,
           Wr1_b, Wr2_b, Wr3_b, Welem_b, Wread_b,
           edge_index, batch):
    sender = edge_index[0]
    receiver = edge_index[1]
    pos_pad = jnp.pad(positions, ((0, 0), (0, 1)))
    w3a = Wr3_a[:, :_C]
    w3b = Wr3_b[:, :_C]
    ae = atomic_energies.reshape(-1, 1)

    r2 = _edge_r2(pos_pad, sender, receiver)
    r2row = r2.reshape(1, _E)

    node_scalars = pl.pallas_call(
        _prep_body,
        out_shape=jax.ShapeDtypeStruct((_N, _C), jnp.float32),
    )(node_attrs, W_embed)

    nt = _E // _T
    full = lambda shape: pl.BlockSpec(shape, lambda i: (0, 0))

    def run_mlp(w1, w2, w3):
        return pl.pallas_call(
            _mlp_body,
            grid=(nt,),
            in_specs=[
                pl.BlockSpec((1, _T), lambda i: (0, i)),
                full((64, 8)), full((64, 64)), full((64, _C)),
            ],
            out_specs=pl.BlockSpec((_T // 8, 8, 128), lambda i: (i, 0, 0)),
            out_shape=jax.ShapeDtypeStruct((_E // 8, 8, 128), jnp.float32),
            compiler_params=pltpu.CompilerParams(
                fuse_transposed_lhs_in_matmul=True),
        )(r2row, w1.T, w2.T, w3)

    m_a = run_mlp(Wr1_a, Wr2_a, w3a)
    m_b = run_mlp(Wr1_b, Wr2_b, w3b)

    agg_a = _scatter_messages(node_scalars, m_a, sender, receiver)

    s1 = pl.pallas_call(
        _update_body,
        out_shape=jax.ShapeDtypeStruct((_N, _C), jnp.float32),
    )(agg_a, node_attrs, Welem_a, node_scalars)

    agg_b = _scatter_messages(s1, m_b, sender, receiver)

    e = pl.pallas_call(
        _final_body,
        out_shape=jax.ShapeDtypeStruct((1, 1), jnp.float32),
    )(agg_b, s1, node_attrs, Welem_b, Wread_a, Wread_b, ae)

    return e[0]


# speedup vs baseline: 2.3982x; 1.0139x over previous
import functools

import jax
import jax.numpy as jnp
from jax import lax
from jax.experimental import pallas as pl
from jax.experimental.pallas import tpu as pltpu
from jax.experimental.pallas import tpu_sc as plsc

_N = 10000
_E = 320000
_C = 32
_R_MAX = 5.0
_INV_AVG = 1.0 / 32.0

_NC = 2
_NS = 16
_NW = _NC * _NS
_EPW = _E // _NW
_K = 1000
_G = _EPW // _K
_KS = 200
_GS = _EPW // _KS
_NPAD = 10240
_RPS = _NPAD // _NS

_sc_mesh = plsc.VectorSubcoreMesh(core_axis_name="c", subcore_axis_name="s")


@functools.partial(
    pl.kernel,
    out_type=jax.ShapeDtypeStruct((_E,), jnp.float32),
    mesh=_sc_mesh,
    scratch_types=[
        pltpu.VMEM((_N, 4), jnp.float32),
        pltpu.VMEM((_K,), jnp.int32),
        pltpu.VMEM((_K,), jnp.int32),
        pltpu.VMEM((_K,), jnp.float32),
    ],
    compiler_params=pltpu.CompilerParams(use_tc_tiling_on_sc=False,
                                         needs_layout_passes=False),
)
def _edge_r2(pos_hbm, snd_hbm, rcv_hbm, r2_hbm, ptbl, isb, irb, r2b):
    wid = lax.axis_index("s") * _NC + lax.axis_index("c")
    pltpu.sync_copy(pos_hbm, ptbl)
    base = wid * _EPW

    def body(g, carry):
        off = base + g * _K
        pltpu.sync_copy(snd_hbm.at[pl.ds(off, _K)], isb)
        pltpu.sync_copy(rcv_hbm.at[pl.ds(off, _K)], irb)

        @plsc.parallel_loop(0, _K // 16, unroll=2)
        def _grp(t):
            sidx = isb[pl.ds(t * 16, 16)]
            ridx = irb[pl.ds(t * 16, 16)]
            r2 = None
            for c in range(3):
                cc = jnp.full((16,), c, jnp.int32)
                d = (plsc.load_gather(ptbl, [ridx, cc])
                     - plsc.load_gather(ptbl, [sidx, cc]))
                r2 = d * d if r2 is None else r2 + d * d
            r2b[pl.ds(t * 16, 16)] = r2

        pltpu.sync_copy(r2b, r2_hbm.at[pl.ds(off, _K)])
        return carry

    lax.fori_loop(0, _G, body, 0)


def _scatter_body(lane_off, tbl_hbm, m_hbm, snd_hbm, rcv_hbm, out_hbm,
                  is0, ir0, is1, ir1, rows0, rows1, mb0, mb1, zbuf, acc,
                  sg0, sg1, sm0, sm1, si0, si1):
    cid = lax.axis_index("c")
    sid = lax.axis_index("s")
    wid = sid * _NC + cid

    zero = jnp.zeros((16,), jnp.float32)

    @plsc.parallel_loop(0, _RPS, unroll=4)
    def _zb(i):
        zbuf[i, pl.ds(0, 16)] = zero
        zbuf[i, pl.ds(16, 16)] = zero

    pltpu.sync_copy(zbuf, acc.at[pl.ds(sid * _RPS, _RPS)])
    plsc.subcore_barrier()

    base = wid * _EPW

    def issue_idx(g, isb, irb, sem):
        off = base + g * _KS
        pltpu.async_copy(snd_hbm.at[pl.ds(off, _KS)], isb, sem)
        pltpu.async_copy(rcv_hbm.at[pl.ds(off, _KS)], irb, sem)

    def drain_idx(isb, irb, sem):
        pltpu.make_async_copy(snd_hbm.at[pl.ds(0, _KS)], isb, sem).wait()
        pltpu.make_async_copy(rcv_hbm.at[pl.ds(0, _KS)], irb, sem).wait()

    def issue_gm(g, isb, rowsb, mbb, semg, semm):
        off = base + g * _KS
        pltpu.async_copy(tbl_hbm.at[isb], rowsb, semg)
        pltpu.async_copy(
            m_hbm.at[pl.ds(off // 8, _KS // 8), :, pl.ds(lane_off, _C)],
            mbb, semm)

    def drain_gm(rowsb, mbb, semg, semm):
        pltpu.make_async_copy(tbl_hbm.at[pl.ds(0, _KS)], rowsb, semg).wait()
        pltpu.make_async_copy(
            m_hbm.at[pl.ds(0, _KS // 8), :, pl.ds(lane_off, _C)],
            mbb, semm).wait()

    def mul_scatter(rowsb, mbb, irb):
        @plsc.parallel_loop(0, _KS // 8, unroll=2)
        def _mul(t):
            for s in range(8):
                rr = t * 8 + s
                rowsb[rr, pl.ds(0, 16)] = rowsb[rr, pl.ds(0, 16)] * mbb[t, s, pl.ds(0, 16)]
                rowsb[rr, pl.ds(16, 16)] = rowsb[rr, pl.ds(16, 16)] * mbb[t, s, pl.ds(16, 16)]

        pltpu.sync_copy(rowsb, acc.at[irb], add=True)

    pltpu.sync_copy(snd_hbm.at[pl.ds(base, _KS)], is0)
    pltpu.sync_copy(rcv_hbm.at[pl.ds(base, _KS)], ir0)
    issue_gm(0, is0, rows0, mb0, sg0, sm0)
    issue_idx(1, is1, ir1, si1)

    def body(j, carry):
        not_last = j < _GS // 2 - 1
        drain_gm(rows0, mb0, sg0, sm0)
        drain_idx(is1, ir1, si1)
        issue_gm(2 * j + 1, is1, rows1, mb1, sg1, sm1)
        mul_scatter(rows0, mb0, ir0)

        @pl.when(not_last)
        def _():
            issue_idx(2 * j + 2, is0, ir0, si0)

        drain_gm(rows1, mb1, sg1, sm1)

        @pl.when(not_last)
        def _():
            drain_idx(is0, ir0, si0)
            issue_gm(2 * j + 2, is0, rows0, mb0, sg0, sm0)

        mul_scatter(rows1, mb1, ir1)

        @pl.when(not_last)
        def _():
            issue_idx(2 * j + 3, is1, ir1, si1)

        return carry

    lax.fori_loop(0, _GS // 2, body, 0)
    plsc.subcore_barrier()
    pltpu.sync_copy(acc.at[pl.ds(sid * _RPS, _RPS)],
                    out_hbm.at[cid, pl.ds(sid * _RPS, _RPS)])


def _make_scatter(lane_off):
    return pl.kernel(
        functools.partial(_scatter_body, lane_off),
        out_type=jax.ShapeDtypeStruct((_NC, _NPAD, _C), jnp.float32),
        mesh=_sc_mesh,
        scratch_types=[
            pltpu.VMEM((_KS,), jnp.int32),
            pltpu.VMEM((_KS,), jnp.int32),
            pltpu.VMEM((_KS,), jnp.int32),
            pltpu.VMEM((_KS,), jnp.int32),
            pltpu.VMEM((_KS, _C), jnp.float32),
            pltpu.VMEM((_KS, _C), jnp.float32),
            pltpu.VMEM((_KS // 8, 8, _C), jnp.float32),
            pltpu.VMEM((_KS // 8, 8, _C), jnp.float32),
            pltpu.VMEM((_RPS, _C), jnp.float32),
            pltpu.VMEM_SHARED((_NPAD, _C), jnp.float32),
            pltpu.SemaphoreType.DMA,
            pltpu.SemaphoreType.DMA,
            pltpu.SemaphoreType.DMA,
            pltpu.SemaphoreType.DMA,
            pltpu.SemaphoreType.DMA,
            pltpu.SemaphoreType.DMA,
        ],
        compiler_params=pltpu.CompilerParams(use_tc_tiling_on_sc=False),
    )


_scatter_a = _make_scatter(0)
_scatter_b = _make_scatter(_C)


def _silu(x):
    return x / (1.0 + jnp.exp(-x))


def _prep_body(na_ref, we_ref, out_ref):
    out_ref[...] = jnp.dot(na_ref[...], we_ref[...],
                           preferred_element_type=jnp.float32)


_T = 6400


def _mlp_body(r2_ref, w1at, w2at, w1bt, w2bt, w3cat, m_ref):
    r2t = r2_ref[...] + 1e-12
    rt = jnp.sqrt(r2t)
    x = rt * (1.0 / _R_MAX)
    x2 = x * x
    x4 = x2 * x2
    x5 = x4 * x
    env = 1.0 - 21.0 * x5 + 35.0 * x5 * x - 15.0 * x5 * x2
    env = jnp.where(x < 1.0, env, 0.0)
    scale = env * (jnp.sqrt(2.0 / _R_MAX) / rt)
    n = 1.0 + lax.broadcasted_iota(jnp.int32, (8, 1), 0).astype(jnp.float32)
    arg = n * ((jnp.pi / _R_MAX) * rt)
    ef = jnp.sin(arg) * scale

    ha = _silu(jnp.dot(w1at[...], ef, preferred_element_type=jnp.float32))
    ha = _silu(jnp.dot(w2at[...], ha, preferred_element_type=jnp.float32))
    hb = _silu(jnp.dot(w1bt[...], ef, preferred_element_type=jnp.float32))
    hb = _silu(jnp.dot(w2bt[...], hb, preferred_element_type=jnp.float32))
    hcat = jnp.concatenate([ha, hb], axis=0)
    mab = lax.dot_general(
        hcat, w3cat[...], (((0,), (0,)), ((), ())),
        preferred_element_type=jnp.float32)
    m_ref[:, :, pl.ds(0, 2 * _C)] = mab.reshape(_T // 8, 8, 2 * _C)


def _update_body(agg_ref, na_ref, wel_ref, ns_ref, out_ref):
    a = agg_ref[...]
    agg = a[0, :_N] + a[1, :_N]
    wel = jnp.dot(na_ref[...], wel_ref[...], preferred_element_type=jnp.float32)
    out_ref[...] = agg * _INV_AVG * wel + ns_ref[...]


def _final_body(aggb_ref, s1_ref, na_ref, welb_ref, wra_ref, wrb_ref, ae_ref,
                out_ref):
    a = aggb_ref[...]
    aggb = a[0, :_N] + a[1, :_N]
    na = na_ref[...]
    s1 = s1_ref[...]
    welb = jnp.dot(na, welb_ref[...], preferred_element_type=jnp.float32)
    s2 = aggb * _INV_AVG * welb + s1
    e0 = jnp.dot(na, ae_ref[...], preferred_element_type=jnp.float32)
    e1 = jnp.dot(s1, wra_ref[...], preferred_element_type=jnp.float32)
    e2 = jnp.dot(_silu(s2), wrb_ref[...], preferred_element_type=jnp.float32)
    out_ref[...] = jnp.sum(e0 + e1 + e2).reshape(1, 1)


def kernel(positions, node_attrs, W_embed, atomic_energies,
           Wr1_a, Wr2_a, Wr3_a, Welem_a, Wread_a,
           Wr1_b, Wr2_b, Wr3_b, Welem_b, Wread_b,
           edge_index, batch):
    sender = edge_index[0]
    receiver = edge_index[1]
    pos_pad = jnp.pad(positions, ((0, 0), (0, 1)))
    w3a = Wr3_a[:, :_C]
    w3b = Wr3_b[:, :_C]
    ae = atomic_energies.reshape(-1, 1)

    r2 = _edge_r2(pos_pad, sender, receiver)
    r2row = r2.reshape(1, _E)

    node_scalars = pl.pallas_call(
        _prep_body,
        out_shape=jax.ShapeDtypeStruct((_N, _C), jnp.float32),
    )(node_attrs, W_embed)

    z = jnp.zeros((64, _C), jnp.float32)
    w3cat = jnp.concatenate(
        [jnp.concatenate([w3a, z], axis=1),
         jnp.concatenate([z, w3b], axis=1)], axis=0)
    nt = _E // _T
    full = lambda shape: pl.BlockSpec(shape, lambda i: (0, 0))
    m_ab = pl.pallas_call(
        _mlp_body,
        grid=(nt,),
        in_specs=[
            pl.BlockSpec((1, _T), lambda i: (0, i)),
            full((64, 8)), full((64, 64)),
            full((64, 8)), full((64, 64)),
            full((128, 2 * _C)),
        ],
        out_specs=pl.BlockSpec((_T // 8, 8, 128), lambda i: (i, 0, 0)),
        out_shape=jax.ShapeDtypeStruct((_E // 8, 8, 128), jnp.float32),
        compiler_params=pltpu.CompilerParams(
            fuse_transposed_lhs_in_matmul=True),
    )(r2row, Wr1_a.T, Wr2_a.T, Wr1_b.T, Wr2_b.T, w3cat)

    agg_a = _scatter_a(node_scalars, m_ab, sender, receiver)

    s1 = pl.pallas_call(
        _update_body,
        out_shape=jax.ShapeDtypeStruct((_N, _C), jnp.float32),
    )(agg_a, node_attrs, Welem_a, node_scalars)

    agg_b = _scatter_b(s1, m_ab, sender, receiver)

    e = pl.pallas_call(
        _final_body,
        out_shape=jax.ShapeDtypeStruct((1, 1), jnp.float32),
    )(agg_b, s1, node_attrs, Welem_b, Wread_a, Wread_b, ae)

    return e[0]
